# baseline (speedup 1.0000x reference)
import functools

import jax
import jax.numpy as jnp
from jax import lax
from jax.experimental import pallas as pl
from jax.experimental.pallas import tpu as pltpu
from jax.experimental.pallas import tpu_sc as plsc


def kernel(indices, logits, feats):
    B = indices.shape[0]
    NR, DL = logits.shape
    DF = feats.shape[1]

    info = plsc.get_sparse_core_info()
    NC, NS = info.num_cores, info.num_subcores
    NW = NC * NS
    b_per_w = B // NW
    FCH = 8
    n_fch = b_per_w // FCH
    OC = 2048
    n_oc = B // OC
    NH = NR // 2

    idx32 = indices.astype(jnp.int32)
    lT = logits.T

    mesh = plsc.VectorSubcoreMesh(core_axis_name="c", subcore_axis_name="s")

    @functools.partial(
        pl.kernel,
        mesh=mesh,
        compiler_params=pltpu.CompilerParams(needs_layout_passes=False),
        out_type=(
            jax.ShapeDtypeStruct((DL, B), jnp.float32),
            jax.ShapeDtypeStruct((B, DF), jnp.float32),
        ),
        scratch_types=[
            pltpu.VMEM((B,), jnp.int32),
            pltpu.VMEM((NR,), jnp.float32),
            pltpu.VMEM((OC,), jnp.float32),
            pltpu.VMEM((OC,), jnp.float32),
            pltpu.VMEM((FCH, DF), jnp.float32),
            pltpu.VMEM((FCH, DF), jnp.float32),
            pltpu.SemaphoreType.DMA,
            pltpu.SemaphoreType.DMA,
            pltpu.SemaphoreType.DMA,
            pltpu.SemaphoreType.DMA,
            pltpu.SemaphoreType.DMA,
        ],
    )
    def gather_all(idx_hbm, lT_hbm, feats_hbm, oT_hbm, of_hbm,
                   idx_v, row_v, outc0_v, outc1_v, fv0, fv1,
                   sem, fsem0, fsem1, osem0, osem1):
        wid = lax.axis_index("s") * NC + lax.axis_index("c")
        pltpu.sync_copy(idx_hbm, idx_v)

        fbase = wid * b_per_w
        fbufs = (fv0, fv1)
        fsems = (fsem0, fsem1)

        def fbody(j2, c):
            for h in range(2):
                j = j2 * 2 + h
                fb, fs = fbufs[h], fsems[h]

                @pl.when(j2 > 0)
                def _():
                    pltpu.make_async_copy(
                        fb, of_hbm.at[pl.ds(fbase, FCH)], fs).wait()

                r0 = fbase + j * FCH
                pltpu.async_copy(
                    feats_hbm.at[idx_v.at[pl.ds(r0, FCH)]], fb, fs).wait()
                pltpu.async_copy(fb, of_hbm.at[pl.ds(r0, FCH)], fs)
            return c

        lax.fori_loop(0, n_fch // 2, fbody, 0)
        pltpu.make_async_copy(fv0, of_hbm.at[pl.ds(fbase, FCH)], fsem0).wait()
        pltpu.make_async_copy(fv1, of_hbm.at[pl.ds(fbase, FCH)], fsem1).wait()

        n_rows = (DL - 1 - wid) // NW + 1
        obufs = (outc0_v, outc1_v)
        osems = (osem0, osem1)

        def rbody(t, c):
            r = wid + NW * t
            pltpu.async_copy(lT_hbm.at[r], row_v, sem).wait()

            def cbody(k2, c2):
                for h in range(2):
                    k = k2 * 2 + h
                    ob, osem = obufs[h], osems[h]

                    @pl.when((t > 0) | (k2 > 0))
                    def _():
                        pltpu.make_async_copy(
                            ob, oT_hbm.at[r, pl.ds(0, OC)], osem).wait()

                    def vbody(v, c3):
                        iis = []
                        for u in range(8):
                            o = pl.multiple_of(k * OC + v * 128 + u * 16, 16)
                            iis.append(idx_v[pl.ds(o, 16)])
                        gs = [plsc.load_gather(row_v, [ii]) for ii in iis]
                        for u in range(8):
                            o = pl.multiple_of(v * 128 + u * 16, 16)
                            ob[pl.ds(o, 16)] = gs[u]
                        return c3

                    lax.fori_loop(0, OC // 128, vbody, 0)
                    pltpu.async_copy(ob, oT_hbm.at[r, pl.ds(k * OC, OC)], osem)
                return c2

            lax.fori_loop(0, n_oc // 2, cbody, 0)
            return c

        lax.fori_loop(0, n_rows, rbody, 0)
        pltpu.make_async_copy(
            outc0_v, oT_hbm.at[wid, pl.ds(0, OC)], osem0).wait()
        pltpu.make_async_copy(
            outc1_v, oT_hbm.at[wid, pl.ds(0, OC)], osem1).wait()

    oT, out_f = gather_all(idx32, lT, feats)
    return (oT.T, out_f)


# speedup vs baseline: 1.1306x; 1.1306x over previous
import functools

import jax
import jax.numpy as jnp
from jax import lax
from jax.experimental import pallas as pl
from jax.experimental.pallas import tpu as pltpu
from jax.experimental.pallas import tpu_sc as plsc


def kernel(indices, logits, feats):
    B = indices.shape[0]
    NR, DL = logits.shape
    DF = feats.shape[1]

    info = plsc.get_sparse_core_info()
    NC, NS = info.num_cores, info.num_subcores
    NW = NC * NS
    b_per_w = B // NW
    FCH = 16
    n_fch = b_per_w // FCH
    OC = 2048
    n_oc = B // OC

    idx32 = indices.astype(jnp.int32)
    lT = logits.T

    mesh = plsc.VectorSubcoreMesh(core_axis_name="c", subcore_axis_name="s")

    @functools.partial(
        pl.kernel,
        mesh=mesh,
        compiler_params=pltpu.CompilerParams(needs_layout_passes=False),
        out_type=(
            jax.ShapeDtypeStruct((DL, B), jnp.float32),
            jax.ShapeDtypeStruct((B, DF), jnp.float32),
        ),
        scratch_types=[
            pltpu.VMEM((B,), jnp.int32),
            pltpu.VMEM((NR,), jnp.float32),
            pltpu.VMEM((OC,), jnp.float32),
            pltpu.VMEM((OC,), jnp.float32),
            pltpu.VMEM((FCH, DF), jnp.float32),
            pltpu.SemaphoreType.DMA,
            pltpu.SemaphoreType.DMA,
            pltpu.SemaphoreType.DMA,
            pltpu.SemaphoreType.DMA,
            pltpu.SemaphoreType.DMA,
        ],
    )
    def gather_all(idx_hbm, lT_hbm, feats_hbm, oT_hbm, of_hbm,
                   idx_v, row_v, outc0_v, outc1_v, fv,
                   sem, fgsem, fwsem, osem0, osem1):
        wid = lax.axis_index("s") * NC + lax.axis_index("c")
        pltpu.sync_copy(idx_hbm, idx_v)

        fbase = wid * b_per_w
        n_rows = (DL - 1 - wid) // NW + 1
        obufs = (outc0_v, outc1_v)
        osems = (osem0, osem1)

        def rbody(t, c):
            r = wid + NW * t
            rowcp = pltpu.async_copy(lT_hbm.at[r], row_v, sem)

            @pl.when(t > 0)
            def _():
                pltpu.make_async_copy(
                    fv, of_hbm.at[pl.ds(fbase, FCH)], fwsem).wait()

            f0 = fbase + t * FCH
            pltpu.async_copy(feats_hbm.at[idx_v.at[pl.ds(f0, FCH)]], fv, fgsem)

            rowcp.wait()

            def cbody(k2, c2):
                for h in range(2):
                    k = k2 * 2 + h
                    ob, osem = obufs[h], osems[h]

                    @pl.when((t > 0) | (k2 > 0))
                    def _():
                        pltpu.make_async_copy(
                            ob, oT_hbm.at[r, pl.ds(0, OC)], osem).wait()

                    def vbody(v, c3):
                        iis = []
                        for u in range(8):
                            o = pl.multiple_of(k * OC + v * 128 + u * 16, 16)
                            iis.append(idx_v[pl.ds(o, 16)])
                        gs = [plsc.load_gather(row_v, [ii]) for ii in iis]
                        for u in range(8):
                            o = pl.multiple_of(v * 128 + u * 16, 16)
                            ob[pl.ds(o, 16)] = gs[u]
                        return c3

                    lax.fori_loop(0, OC // 128, vbody, 0)
                    pltpu.async_copy(ob, oT_hbm.at[r, pl.ds(k * OC, OC)], osem)
                return c2

            lax.fori_loop(0, n_oc // 2, cbody, 0)

            pltpu.make_async_copy(
                feats_hbm.at[idx_v.at[pl.ds(f0, FCH)]], fv, fgsem).wait()
            pltpu.async_copy(fv, of_hbm.at[pl.ds(f0, FCH)], fwsem)
            return c

        lax.fori_loop(0, n_rows, rbody, 0)

        def ftail(j, c):
            pltpu.make_async_copy(
                fv, of_hbm.at[pl.ds(fbase, FCH)], fwsem).wait()
            f0 = fbase + j * FCH
            pltpu.async_copy(
                feats_hbm.at[idx_v.at[pl.ds(f0, FCH)]], fv, fgsem).wait()
            pltpu.async_copy(fv, of_hbm.at[pl.ds(f0, FCH)], fwsem)
            return c

        lax.fori_loop(n_rows, n_fch, ftail, 0)
        pltpu.make_async_copy(fv, of_hbm.at[pl.ds(fbase, FCH)], fwsem).wait()
        pltpu.make_async_copy(
            outc0_v, oT_hbm.at[wid, pl.ds(0, OC)], osem0).wait()
        pltpu.make_async_copy(
            outc1_v, oT_hbm.at[wid, pl.ds(0, OC)], osem1).wait()

    oT, out_f = gather_all(idx32, lT, feats)
    return (oT.T, out_f)
